# Initial kernel scaffold; baseline (speedup 1.0000x reference)
import jax
import jax.numpy as jnp
from jax.experimental import pallas as pl


def kernel(pro1_x, pro2_x, mas1_straight, mas1_flipped, mas2_straight, mas2_flipped, params, pro1_edge_index, pro1_batch, pro2_edge_index, pro2_batch):
    raise NotImplementedError("write your pallas kernel here")


# speedup vs baseline: 18.1658x; 18.1658x over previous
import functools
import math

import jax
import jax.numpy as jnp
from jax import lax
from jax.experimental import pallas as pl
from jax.experimental.pallas import tpu as pltpu
from jax.experimental.pallas import tpu_sc as plsc

N = 10000
E = 320000
D = 128
B = 64
L = 50
DESC = 80
TD = 31
DM = 32
NH = 4
FF = 128
OUT = 128
NL = 2
HD = DM // NH
S2 = 2 * L

NC = 2
NS = 16
NW = NC * NS
CW = 125
CPW = E // NW // CW
RPT8 = 640
RPT_LAST = N - (NS - 1) * RPT8

RB = 2000
NRB = N // RB

def _sc_mesh():
    return plsc.VectorSubcoreMesh(
        core_axis_name="c", subcore_axis_name="s", num_cores=NC,
        num_subcores=NS)


def _leaky(v):
    return jnp.where(v >= 0, v, 0.01 * v)


def _deg_body(dst1, dst2, ones_n, degp, idx_v, ones_v, h1, h2):
    cid = lax.axis_index("c")
    sid = lax.axis_index("s")
    wid = sid * NC + cid
    for i in range(8):
        ones_v[pl.ds(i * 16, 16)] = jnp.ones((16,), jnp.float32)

    @pl.when(sid == 0)
    def _():
        pltpu.sync_copy(ones_n, h1)
        pltpu.sync_copy(ones_n, h2)

    plsc.subcore_barrier()
    for dst, h in ((dst1, h1), (dst2, h2)):
        pltpu.sync_copy(dst.at[pl.ds(wid * CPW, CPW)], idx_v)

        def body(j, carry, h=h):
            pltpu.sync_copy(ones_v.at[pl.ds(0, CW)], h.at[idx_v.at[j]],
                            add=True)
            return carry

        lax.fori_loop(0, CPW, body, 0)
    plsc.subcore_barrier()

    @pl.when(sid == 0)
    def _():
        pltpu.sync_copy(h1, degp.at[0, cid])
        pltpu.sync_copy(h2, degp.at[1, cid])


def _scale_body(degp_ref, x1_ref, x2_ref, u1_ref, u2_ref):
    d1 = degp_ref[0, 0, 0, 0, :] + degp_ref[0, 1, 0, 0, :]
    d2 = degp_ref[1, 0, 0, 0, :] + degp_ref[1, 1, 0, 0, :]
    dis1 = lax.rsqrt(d1 - 1.0)
    dis2 = lax.rsqrt(d2 - 1.0)
    u1_ref[...] = x1_ref[...] * dis1[:, None]
    u2_ref[...] = x2_ref[...] * dis2[:, None]


def _scale_call(degp5, x1, x2):
    return pl.pallas_call(
        _scale_body,
        grid=(NRB,),
        in_specs=[
            pl.BlockSpec((2, NC, 1, 1, RB), lambda i: (0, 0, i, 0, 0)),
            pl.BlockSpec((RB, D), lambda i: (i, 0)),
            pl.BlockSpec((RB, D), lambda i: (i, 0)),
        ],
        out_specs=[
            pl.BlockSpec((RB, D), lambda i: (i, 0)),
            pl.BlockSpec((RB, D), lambda i: (i, 0)),
        ],
        out_shape=[
            jax.ShapeDtypeStruct((N, D), jnp.float32),
            jax.ShapeDtypeStruct((N, D), jnp.float32),
        ],
    )(degp5, x1, x2)


def _agg_body(u1, u2, s1, d1, s2, d2, Sp, sidx_v, didx_v, rows_v, S_sh,
              sem):
    cid = lax.axis_index("c")
    sid = lax.axis_index("s")
    wid = sid * NC + cid
    base = pl.multiple_of(sid * RPT8, 8)
    for g, (u, si, di) in enumerate(((u1, s1, d1), (u2, s2, d2))):
        @pl.when(sid < NS - 1)
        def _(u=u):
            pltpu.sync_copy(u.at[pl.ds(base, RPT8)],
                            S_sh.at[pl.ds(base, RPT8)])

        @pl.when(sid == NS - 1)
        def _(u=u):
            pltpu.sync_copy(u.at[pl.ds((NS - 1) * RPT8, RPT_LAST)],
                            S_sh.at[pl.ds((NS - 1) * RPT8, RPT_LAST)])

        pltpu.sync_copy(si.at[pl.ds(wid * CPW, CPW)], sidx_v)
        pltpu.sync_copy(di.at[pl.ds(wid * CPW, CPW)], didx_v)
        plsc.subcore_barrier()

        def body(j, carry, u=u):
            pltpu.async_copy(u.at[sidx_v.at[j]], rows_v, sem).wait()
            pltpu.sync_copy(rows_v, S_sh.at[didx_v.at[j]], add=True)
            return carry

        lax.fori_loop(0, CPW, body, 0)
        plsc.subcore_barrier()

        @pl.when(sid < NS - 1)
        def _(g=g):
            pltpu.sync_copy(S_sh.at[pl.ds(base, RPT8)],
                            Sp.at[g, cid, pl.ds(base, RPT8)])

        @pl.when(sid == NS - 1)
        def _(g=g):
            pltpu.sync_copy(S_sh.at[pl.ds((NS - 1) * RPT8, RPT_LAST)],
                            Sp.at[g, cid, pl.ds((NS - 1) * RPT8, RPT_LAST)])


@functools.cache
def _deg_kernel():
    return pl.kernel(
        _deg_body,
        out_type=jax.ShapeDtypeStruct((2, NC, N), jnp.float32),
        mesh=_sc_mesh(),
        scratch_types=[
            pltpu.VMEM((CPW, CW), jnp.int32),
            pltpu.VMEM((128,), jnp.float32),
            pltpu.VMEM_SHARED((N,), jnp.float32),
            pltpu.VMEM_SHARED((N,), jnp.float32),
        ],
    )


@functools.cache
def _agg_kernel():
    return pl.kernel(
        _agg_body,
        out_type=jax.ShapeDtypeStruct((2, NC, N, D), jnp.float32),
        mesh=_sc_mesh(),
        scratch_types=[
            pltpu.VMEM((CPW, CW), jnp.int32),
            pltpu.VMEM((CPW, CW), jnp.int32),
            pltpu.VMEM((CW, D), jnp.float32),
            pltpu.VMEM_SHARED((N, D), jnp.float32),
            pltpu.SemaphoreType.DMA,
        ],
    )


def _ln(x, g, b):
    m = x.mean(-1, keepdims=True)
    v = ((x - m) ** 2).mean(-1, keepdims=True)
    return (x - m) / jnp.sqrt(v + 1e-5) * g + b


def _encoder(x, l, WqkvT, bqkv, WoT, bo, ln1g, ln1b, W1T, b1, W2T, b2,
             ln2g, ln2b):
    qkv = jnp.dot(x, WqkvT[l], preferred_element_type=jnp.float32) + bqkv[l]
    acc = jnp.zeros((S2, DM), jnp.float32)
    for h in range(NH):
        q = qkv[:, h * HD:(h + 1) * HD]
        k = qkv[:, DM + h * HD:DM + (h + 1) * HD]
        v = qkv[:, 2 * DM + h * HD:2 * DM + (h + 1) * HD]
        s = lax.dot_general(q, k, (((1,), (1,)), ((), ())),
                            preferred_element_type=jnp.float32)
        s = s / math.sqrt(float(HD))
        s = s - jnp.max(s, axis=-1, keepdims=True)
        p = jnp.exp(s)
        p = p / jnp.sum(p, axis=-1, keepdims=True)
        oh = jnp.dot(p, v, preferred_element_type=jnp.float32)
        acc = acc + jnp.dot(oh, WoT[l, h * HD:(h + 1) * HD, :],
                            preferred_element_type=jnp.float32)
    x = _ln(x + acc + bo[l], ln1g[l], ln1b[l])
    f = jnp.maximum(
        jnp.dot(x, W1T[l], preferred_element_type=jnp.float32) + b1[l], 0.0)
    f = jnp.dot(f, W2T[l], preferred_element_type=jnp.float32) + b2[l]
    return _ln(x + f, ln2g[l], ln2b[l])


def _tr_body(x1_ref, x2_ref, WaugT_ref, baug_ref, WqkvT_ref, bqkv_ref,
             WoT_ref, bo_ref, ln1g_ref, ln1b_ref, W1T_ref, b1_ref, W2T_ref,
             b2_ref, ln2g_ref, ln2b_ref, m1_ref, m2_ref):
    WaugT = WaugT_ref[...]
    baug = baug_ref[...]
    enc_args = (WqkvT_ref[...], bqkv_ref[...], WoT_ref[...], bo_ref[...],
                ln1g_ref[...], ln1b_ref[...], W1T_ref[...], b1_ref[...],
                W2T_ref[...], b2_ref[...], ln2g_ref[...], ln2b_ref[...])
    for x_ref, m_ref in ((x1_ref, m1_ref), (x2_ref, m2_ref)):
        x = jnp.dot(x_ref[0], WaugT, preferred_element_type=jnp.float32)
        x = x + baug
        for l in range(NL):
            x = _encoder(x, l, *enc_args)
        m_ref[0, 0, :] = jnp.mean(x, axis=0)


def _tr_call(x1aug, x2aug, WaugT, baug, stk):
    (WqkvT, bqkv, WoT, bo, ln1g, ln1b, W1T, b1, W2T, b2, ln2g, ln2b) = stk
    whole = lambda a: pl.BlockSpec(a.shape, lambda i: (0,) * a.ndim)
    m1, m2 = pl.pallas_call(
        _tr_body,
        grid=(B,),
        in_specs=[
            pl.BlockSpec((1, S2, DESC + 1), lambda i: (i, 0, 0)),
            pl.BlockSpec((1, S2, DESC + 1), lambda i: (i, 0, 0)),
            whole(WaugT), whole(baug), whole(WqkvT), whole(bqkv),
            whole(WoT), whole(bo), whole(ln1g), whole(ln1b), whole(W1T),
            whole(b1), whole(W2T), whole(b2), whole(ln2g), whole(ln2b),
        ],
        out_specs=[
            pl.BlockSpec((1, 1, DM), lambda i: (i, 0, 0)),
            pl.BlockSpec((1, 1, DM), lambda i: (i, 0, 0)),
        ],
        out_shape=[
            jax.ShapeDtypeStruct((B, 1, DM), jnp.float32),
            jax.ShapeDtypeStruct((B, 1, DM), jnp.float32),
        ],
    )(x1aug, x2aug, WaugT, baug, WqkvT, bqkv, WoT, bo, ln1g, ln1b, W1T, b1,
      W2T, b2, ln2g, ln2b)
    return m1[:, 0, :], m2[:, 0, :]


def _final_body(degp_ref, u1_ref, u2_ref, s10_ref, s11_ref, s20_ref,
                s21_ref, bt1_ref, bt2_ref, w1t_ref, b1_ref, w2t_ref, b2_ref,
                fc1t_ref, fc1b_ref, fc2t_ref, fc2b_ref, f1_ref, f2_ref,
                f3_ref, f4_ref, fb_ref, m1_ref, m2_ref, out_ref,
                acc1, cnt1, acc2, cnt2):
    i = pl.program_id(0)

    @pl.when(i == 0)
    def _():
        acc1[...] = jnp.zeros_like(acc1)
        cnt1[...] = jnp.zeros_like(cnt1)
        acc2[...] = jnp.zeros_like(acc2)
        cnt2[...] = jnp.zeros_like(cnt2)

    iota = lax.broadcasted_iota(jnp.int32, (B, RB), 0)
    for g, (u_ref, sa_ref, sb_ref, bt_ref, wt_ref, wb_ref, acc, cnt) in (
            (0, (u1_ref, s10_ref, s11_ref, bt1_ref, w1t_ref, b1_ref, acc1,
                 cnt1)),
            (1, (u2_ref, s20_ref, s21_ref, bt2_ref, w2t_ref, b2_ref, acc2,
                 cnt2))):
        dg = degp_ref[g, 0, 0, 0, :] + degp_ref[g, 1, 0, 0, :]
        dis = lax.rsqrt(dg - 1.0)
        agg = (sa_ref[...] + sb_ref[...] - u_ref[...]) * dis[:, None]
        h = _leaky(jnp.dot(agg, wt_ref[...],
                           preferred_element_type=jnp.float32) + wb_ref[...])
        bt = bt_ref[0, 0, :]
        mask = (bt[None, :] == iota).astype(jnp.float32)
        acc[...] += jnp.dot(mask, h, preferred_element_type=jnp.float32)
        cnt[...] += jnp.sum(mask, axis=1, keepdims=True)

    @pl.when(i == NRB - 1)
    def _():
        p1 = acc1[...] / jnp.maximum(cnt1[...], 1.0)
        p2 = acc2[...] / jnp.maximum(cnt2[...], 1.0)
        x1p = _leaky(jnp.dot(p1, fc1t_ref[...],
                             preferred_element_type=jnp.float32)
                     + fc1b_ref[...])
        x2p = _leaky(jnp.dot(p2, fc2t_ref[...],
                             preferred_element_type=jnp.float32)
                     + fc2b_ref[...])
        out = (jnp.dot(x1p, f1_ref[...], preferred_element_type=jnp.float32)
               + jnp.dot(x2p, f2_ref[...],
                         preferred_element_type=jnp.float32)
               + jnp.dot(m1_ref[...], f3_ref[...],
                         preferred_element_type=jnp.float32)
               + jnp.dot(m2_ref[...], f4_ref[...],
                         preferred_element_type=jnp.float32))
        out_ref[...] = out + fb_ref[...]


def _final_call(degp5, u1, u2, s10, s11, s20, s21, bt1, bt2, w1t, b1, w2t,
                b2, fc1t, fc1b, fc2t, fc2b, f1, f2, f3, f4, fb, m1, m2):
    whole = lambda a: pl.BlockSpec(a.shape, lambda i: (0,) * a.ndim)
    rowblk = pl.BlockSpec((RB, D), lambda i: (i, 0))
    btblk = pl.BlockSpec((1, 1, RB), lambda i: (i, 0, 0))
    return pl.pallas_call(
        _final_body,
        grid=(NRB,),
        in_specs=[
            pl.BlockSpec((2, NC, 1, 1, RB), lambda i: (0, 0, i, 0, 0)),
            rowblk, rowblk, rowblk, rowblk, rowblk, rowblk,
            btblk, btblk, whole(w1t), whole(b1), whole(w2t),
            whole(b2), whole(fc1t), whole(fc1b), whole(fc2t), whole(fc2b),
            whole(f1), whole(f2), whole(f3), whole(f4), whole(fb),
            whole(m1), whole(m2),
        ],
        out_specs=pl.BlockSpec((B, 1), lambda i: (0, 0)),
        out_shape=jax.ShapeDtypeStruct((B, 1), jnp.float32),
        scratch_shapes=[
            pltpu.VMEM((B, D), jnp.float32),
            pltpu.VMEM((B, 1), jnp.float32),
            pltpu.VMEM((B, D), jnp.float32),
            pltpu.VMEM((B, 1), jnp.float32),
        ],
    )(degp5, u1, u2, s10, s11, s20, s21, bt1, bt2, w1t, b1, w2t, b2, fc1t,
      fc1b, fc2t, fc2b, f1, f2, f3, f4, fb, m1, m2)


def kernel(pro1_x, pro2_x, mas1_straight, mas1_flipped, mas2_straight,
           mas2_flipped, params, pro1_edge_index, pro1_batch,
           pro2_edge_index, pro2_batch):
    p = params
    ei1 = pro1_edge_index.astype(jnp.int32)
    ei2 = pro2_edge_index.astype(jnp.int32)
    src1 = ei1[0].reshape(E // CW, CW)
    dst1 = ei1[1].reshape(E // CW, CW)
    src2 = ei2[0].reshape(E // CW, CW)
    dst2 = ei2[1].reshape(E // CW, CW)
    bt1 = pro1_batch.astype(jnp.int32).reshape(NRB, 1, RB)
    bt2 = pro2_batch.astype(jnp.int32).reshape(NRB, 1, RB)
    ones_n = jnp.ones((N,), jnp.float32)

    degp = _deg_kernel()(dst1, dst2, ones_n)
    degp5 = degp.reshape(2, NC, NRB, 1, RB)
    u1, u2 = _scale_call(degp5, pro1_x, pro2_x)
    Sp = _agg_kernel()(u1, u2, src1, dst1, src2, dst2)

    ind = jnp.concatenate(
        [jnp.ones((B, L, 1), jnp.float32), jnp.zeros((B, L, 1), jnp.float32)],
        axis=1)
    x1aug = jnp.concatenate(
        [jnp.concatenate([mas1_straight, mas1_flipped], axis=1), ind],
        axis=-1)
    x2aug = jnp.concatenate(
        [jnp.concatenate([mas2_straight, mas2_flipped], axis=1), ind],
        axis=-1)
    Waug = jnp.zeros((DM, DESC + 1), jnp.float32)
    Waug = Waug.at[:TD, :DESC].set(p['red_W'])
    Waug = Waug.at[TD, DESC].set(1.0)
    baug = jnp.concatenate([p['red_b'], jnp.zeros((1,), jnp.float32)])
    stk = tuple(
        jnp.stack([p['t%d_%s' % (l, n)] for l in range(NL)])
        for n in ('Wqkv', 'bqkv', 'Wo', 'bo', 'ln1g', 'ln1b', 'W1', 'b1',
                  'W2', 'b2', 'ln2g', 'ln2b'))
    (WqkvS, bqkvS, WoS, boS, ln1gS, ln1bS, W1S, b1S, W2S, b2S, ln2gS,
     ln2bS) = stk
    m1, m2 = _tr_call(
        x1aug, x2aug, Waug.T, baug,
        (WqkvS.transpose(0, 2, 1), bqkvS, WoS.transpose(0, 2, 1), boS,
         ln1gS, ln1bS, W1S.transpose(0, 2, 1), b1S, W2S.transpose(0, 2, 1),
         b2S, ln2gS, ln2bS))

    fw = p['fin_W']
    out = _final_call(
        degp5, u1, u2, Sp[0, 0], Sp[0, 1], Sp[1, 0], Sp[1, 1], bt1, bt2,
        p['gcn1_W'].T, p['gcn1_b'], p['gcn2_W'].T, p['gcn2_b'],
        p['fc1_W'].T, p['fc1_b'], p['fc2_W'].T, p['fc2_b'],
        fw[:, :OUT].T, fw[:, OUT:2 * OUT].T,
        fw[:, 2 * OUT:2 * OUT + DM].T, fw[:, 2 * OUT + DM:].T,
        p['fin_b'], m1, m2)
    return out


# speedup vs baseline: 25.5170x; 1.4047x over previous
import functools
import math

import jax
import jax.numpy as jnp
from jax import lax
from jax.experimental import pallas as pl
from jax.experimental.pallas import tpu as pltpu
from jax.experimental.pallas import tpu_sc as plsc

N = 10000
E = 320000
D = 128
B = 64
L = 50
DESC = 80
TD = 31
DM = 32
NH = 4
FF = 128
OUT = 128
NL = 2
HD = DM // NH
S2 = 2 * L

NC = 2
NS = 16
NW = NC * NS
CW = 125
CPW = E // NW // CW
RPT8 = 640
RPT_LAST = N - (NS - 1) * RPT8

RB = 2000
NRB = N // RB

def _sc_mesh():
    return plsc.VectorSubcoreMesh(
        core_axis_name="c", subcore_axis_name="s", num_cores=NC,
        num_subcores=NS)


def _leaky(v):
    return jnp.where(v >= 0, v, 0.01 * v)


def _deg_body(dst1, dst2, ones_n, degp, idx_v, ones_v, h1, h2):
    cid = lax.axis_index("c")
    sid = lax.axis_index("s")
    wid = sid * NC + cid
    for i in range(8):
        ones_v[pl.ds(i * 16, 16)] = jnp.ones((16,), jnp.float32)

    @pl.when(sid == 0)
    def _():
        pltpu.sync_copy(ones_n, h1)
        pltpu.sync_copy(ones_n, h2)

    plsc.subcore_barrier()
    for dst, h in ((dst1, h1), (dst2, h2)):
        pltpu.sync_copy(dst.at[pl.ds(wid * CPW, CPW)], idx_v)

        def body(j, carry, h=h):
            pltpu.sync_copy(ones_v.at[pl.ds(0, CW)], h.at[idx_v.at[j]],
                            add=True)
            return carry

        lax.fori_loop(0, CPW, body, 0)
    plsc.subcore_barrier()

    @pl.when(sid == 0)
    def _():
        pltpu.sync_copy(h1, degp.at[0, cid])
        pltpu.sync_copy(h2, degp.at[1, cid])


def _scale_body(degp_ref, x1_ref, x2_ref, u1_ref, u2_ref):
    d1 = degp_ref[0, 0, 0, 0, :] + degp_ref[0, 1, 0, 0, :]
    d2 = degp_ref[1, 0, 0, 0, :] + degp_ref[1, 1, 0, 0, :]
    dis1 = lax.rsqrt(d1 - 1.0)
    dis2 = lax.rsqrt(d2 - 1.0)
    u1_ref[...] = x1_ref[...] * dis1[:, None]
    u2_ref[...] = x2_ref[...] * dis2[:, None]


def _scale_call(degp5, x1, x2):
    return pl.pallas_call(
        _scale_body,
        grid=(NRB,),
        in_specs=[
            pl.BlockSpec((2, NC, 1, 1, RB), lambda i: (0, 0, i, 0, 0)),
            pl.BlockSpec((RB, D), lambda i: (i, 0)),
            pl.BlockSpec((RB, D), lambda i: (i, 0)),
        ],
        out_specs=[
            pl.BlockSpec((RB, D), lambda i: (i, 0)),
            pl.BlockSpec((RB, D), lambda i: (i, 0)),
        ],
        out_shape=[
            jax.ShapeDtypeStruct((N, D), jnp.float32),
            jax.ShapeDtypeStruct((N, D), jnp.float32),
        ],
    )(degp5, x1, x2)


def _agg_body(u1, u2, s1, d1, s2, d2, Sp, sidx_v, didx_v, rows_v, S_sh,
              sem):
    cid = lax.axis_index("c")
    sid = lax.axis_index("s")
    wid = sid * NC + cid
    base = pl.multiple_of(sid * RPT8, 8)
    for g, (u, si, di) in enumerate(((u1, s1, d1), (u2, s2, d2))):
        @pl.when(sid < NS - 1)
        def _(u=u):
            pltpu.sync_copy(u.at[pl.ds(base, RPT8)],
                            S_sh.at[pl.ds(base, RPT8)])

        @pl.when(sid == NS - 1)
        def _(u=u):
            pltpu.sync_copy(u.at[pl.ds((NS - 1) * RPT8, RPT_LAST)],
                            S_sh.at[pl.ds((NS - 1) * RPT8, RPT_LAST)])

        pltpu.sync_copy(si.at[pl.ds(wid * CPW, CPW)], sidx_v)
        pltpu.sync_copy(di.at[pl.ds(wid * CPW, CPW)], didx_v)
        plsc.subcore_barrier()

        def body(j, carry, u=u):
            pltpu.async_copy(u.at[sidx_v.at[j]], rows_v, sem).wait()
            pltpu.sync_copy(rows_v, S_sh.at[didx_v.at[j]], add=True)
            return carry

        lax.fori_loop(0, CPW, body, 0)
        plsc.subcore_barrier()

        @pl.when(sid < NS - 1)
        def _(g=g):
            pltpu.sync_copy(S_sh.at[pl.ds(base, RPT8)],
                            Sp.at[g, cid, pl.ds(base, RPT8)])

        @pl.when(sid == NS - 1)
        def _(g=g):
            pltpu.sync_copy(S_sh.at[pl.ds((NS - 1) * RPT8, RPT_LAST)],
                            Sp.at[g, cid, pl.ds((NS - 1) * RPT8, RPT_LAST)])


@functools.cache
def _deg_kernel():
    return pl.kernel(
        _deg_body,
        out_type=jax.ShapeDtypeStruct((2, NC, N), jnp.float32),
        mesh=_sc_mesh(),
        scratch_types=[
            pltpu.VMEM((CPW, CW), jnp.int32),
            pltpu.VMEM((128,), jnp.float32),
            pltpu.VMEM_SHARED((N,), jnp.float32),
            pltpu.VMEM_SHARED((N,), jnp.float32),
        ],
    )


@functools.cache
def _agg_kernel():
    return pl.kernel(
        _agg_body,
        out_type=jax.ShapeDtypeStruct((2, NC, N, D), jnp.float32),
        mesh=_sc_mesh(),
        scratch_types=[
            pltpu.VMEM((CPW, CW), jnp.int32),
            pltpu.VMEM((CPW, CW), jnp.int32),
            pltpu.VMEM((CW, D), jnp.float32),
            pltpu.VMEM_SHARED((N, D), jnp.float32),
            pltpu.SemaphoreType.DMA,
        ],
    )


def _ln(x, g, b):
    m = x.mean(-1, keepdims=True)
    v = ((x - m) ** 2).mean(-1, keepdims=True)
    return (x - m) / jnp.sqrt(v + 1e-5) * g + b


GTB = 4


def _encoder(x, l, hmasks, WqT, WkT, WvT, bq, bk, bv, WoT, bo, ln1g, ln1b,
             W1T, b1, W2T, b2, ln2g, ln2b):
    q = jnp.dot(x, WqT[l], preferred_element_type=jnp.float32) + bq[l]
    k = jnp.dot(x, WkT[l], preferred_element_type=jnp.float32) + bk[l]
    v = jnp.dot(x, WvT[l], preferred_element_type=jnp.float32) + bv[l]
    acc = jnp.zeros((S2, DM), jnp.float32)
    for h in range(NH):
        s = lax.dot_general(q, k * hmasks[h], (((1,), (1,)), ((), ())),
                            preferred_element_type=jnp.float32)
        s = s * (1.0 / math.sqrt(float(HD)))
        s = s - jnp.max(s, axis=-1, keepdims=True)
        p = jnp.exp(s)
        p = p / jnp.sum(p, axis=-1, keepdims=True)
        acc = acc + jnp.dot(p, v * hmasks[h],
                            preferred_element_type=jnp.float32)
    o = jnp.dot(acc, WoT[l], preferred_element_type=jnp.float32)
    x = _ln(x + o + bo[l], ln1g[l], ln1b[l])
    f = jnp.maximum(
        jnp.dot(x, W1T[l], preferred_element_type=jnp.float32) + b1[l], 0.0)
    f = jnp.dot(f, W2T[l], preferred_element_type=jnp.float32) + b2[l]
    return _ln(x + f, ln2g[l], ln2b[l])


def _tr_body(x1_ref, x2_ref, WaugT_ref, baug_ref, WqT_ref, WkT_ref, WvT_ref,
             bq_ref, bk_ref, bv_ref, WoT_ref, bo_ref, ln1g_ref, ln1b_ref,
             W1T_ref, b1_ref, W2T_ref, b2_ref, ln2g_ref, ln2b_ref, m1_ref,
             m2_ref):
    WaugT = WaugT_ref[...]
    baug = baug_ref[...]
    lane = lax.broadcasted_iota(jnp.int32, (1, DM), 1)
    hmasks = [((lane >= h * HD) & (lane < (h + 1) * HD)).astype(jnp.float32)
              for h in range(NH)]
    enc_args = (hmasks, WqT_ref[...], WkT_ref[...], WvT_ref[...],
                bq_ref[...], bk_ref[...], bv_ref[...], WoT_ref[...],
                bo_ref[...], ln1g_ref[...], ln1b_ref[...], W1T_ref[...],
                b1_ref[...], W2T_ref[...], b2_ref[...], ln2g_ref[...],
                ln2b_ref[...])
    for gi in range(GTB):
        for x_ref, m_ref in ((x1_ref, m1_ref), (x2_ref, m2_ref)):
            x = jnp.dot(x_ref[gi], WaugT, preferred_element_type=jnp.float32)
            x = x + baug
            for l in range(NL):
                x = _encoder(x, l, *enc_args)
            m_ref[gi, 0, :] = jnp.mean(x, axis=0)


def _tr_call(x1aug, x2aug, WaugT, baug, stk):
    (WqT, WkT, WvT, bq, bk, bv, WoT, bo, ln1g, ln1b, W1T, b1, W2T, b2,
     ln2g, ln2b) = stk
    whole = lambda a: pl.BlockSpec(a.shape, lambda i: (0,) * a.ndim)
    m1, m2 = pl.pallas_call(
        _tr_body,
        grid=(B // GTB,),
        in_specs=[
            pl.BlockSpec((GTB, S2, DESC + 1), lambda i: (i, 0, 0)),
            pl.BlockSpec((GTB, S2, DESC + 1), lambda i: (i, 0, 0)),
            whole(WaugT), whole(baug), whole(WqT), whole(WkT), whole(WvT),
            whole(bq), whole(bk), whole(bv), whole(WoT), whole(bo),
            whole(ln1g), whole(ln1b), whole(W1T), whole(b1), whole(W2T),
            whole(b2), whole(ln2g), whole(ln2b),
        ],
        out_specs=[
            pl.BlockSpec((GTB, 1, DM), lambda i: (i, 0, 0)),
            pl.BlockSpec((GTB, 1, DM), lambda i: (i, 0, 0)),
        ],
        out_shape=[
            jax.ShapeDtypeStruct((B, 1, DM), jnp.float32),
            jax.ShapeDtypeStruct((B, 1, DM), jnp.float32),
        ],
    )(x1aug, x2aug, WaugT, baug, WqT, WkT, WvT, bq, bk, bv, WoT, bo, ln1g,
      ln1b, W1T, b1, W2T, b2, ln2g, ln2b)
    return m1[:, 0, :], m2[:, 0, :]


def _final_body(degp_ref, u1_ref, u2_ref, s10_ref, s11_ref, s20_ref,
                s21_ref, bt1_ref, bt2_ref, w1t_ref, b1_ref, w2t_ref, b2_ref,
                fc1t_ref, fc1b_ref, fc2t_ref, fc2b_ref, f1_ref, f2_ref,
                f3_ref, f4_ref, fb_ref, m1_ref, m2_ref, out_ref,
                acc1, cnt1, acc2, cnt2):
    i = pl.program_id(0)

    @pl.when(i == 0)
    def _():
        acc1[...] = jnp.zeros_like(acc1)
        cnt1[...] = jnp.zeros_like(cnt1)
        acc2[...] = jnp.zeros_like(acc2)
        cnt2[...] = jnp.zeros_like(cnt2)

    iota = lax.broadcasted_iota(jnp.int32, (B, RB), 0)
    for g, (u_ref, sa_ref, sb_ref, bt_ref, wt_ref, wb_ref, acc, cnt) in (
            (0, (u1_ref, s10_ref, s11_ref, bt1_ref, w1t_ref, b1_ref, acc1,
                 cnt1)),
            (1, (u2_ref, s20_ref, s21_ref, bt2_ref, w2t_ref, b2_ref, acc2,
                 cnt2))):
        dg = degp_ref[g, 0, 0, 0, :] + degp_ref[g, 1, 0, 0, :]
        dis = lax.rsqrt(dg - 1.0)
        agg = (sa_ref[...] + sb_ref[...] - u_ref[...]) * dis[:, None]
        h = _leaky(jnp.dot(agg, wt_ref[...],
                           preferred_element_type=jnp.float32) + wb_ref[...])
        bt = bt_ref[0, 0, :]
        mask = (bt[None, :] == iota).astype(jnp.float32)
        acc[...] += jnp.dot(mask, h, preferred_element_type=jnp.float32)
        cnt[...] += jnp.sum(mask, axis=1, keepdims=True)

    @pl.when(i == NRB - 1)
    def _():
        p1 = acc1[...] / jnp.maximum(cnt1[...], 1.0)
        p2 = acc2[...] / jnp.maximum(cnt2[...], 1.0)
        x1p = _leaky(jnp.dot(p1, fc1t_ref[...],
                             preferred_element_type=jnp.float32)
                     + fc1b_ref[...])
        x2p = _leaky(jnp.dot(p2, fc2t_ref[...],
                             preferred_element_type=jnp.float32)
                     + fc2b_ref[...])
        out = (jnp.dot(x1p, f1_ref[...], preferred_element_type=jnp.float32)
               + jnp.dot(x2p, f2_ref[...],
                         preferred_element_type=jnp.float32)
               + jnp.dot(m1_ref[...], f3_ref[...],
                         preferred_element_type=jnp.float32)
               + jnp.dot(m2_ref[...], f4_ref[...],
                         preferred_element_type=jnp.float32))
        out_ref[...] = out + fb_ref[...]


def _final_call(degp5, u1, u2, s10, s11, s20, s21, bt1, bt2, w1t, b1, w2t,
                b2, fc1t, fc1b, fc2t, fc2b, f1, f2, f3, f4, fb, m1, m2):
    whole = lambda a: pl.BlockSpec(a.shape, lambda i: (0,) * a.ndim)
    rowblk = pl.BlockSpec((RB, D), lambda i: (i, 0))
    btblk = pl.BlockSpec((1, 1, RB), lambda i: (i, 0, 0))
    return pl.pallas_call(
        _final_body,
        grid=(NRB,),
        in_specs=[
            pl.BlockSpec((2, NC, 1, 1, RB), lambda i: (0, 0, i, 0, 0)),
            rowblk, rowblk, rowblk, rowblk, rowblk, rowblk,
            btblk, btblk, whole(w1t), whole(b1), whole(w2t),
            whole(b2), whole(fc1t), whole(fc1b), whole(fc2t), whole(fc2b),
            whole(f1), whole(f2), whole(f3), whole(f4), whole(fb),
            whole(m1), whole(m2),
        ],
        out_specs=pl.BlockSpec((B, 1), lambda i: (0, 0)),
        out_shape=jax.ShapeDtypeStruct((B, 1), jnp.float32),
        scratch_shapes=[
            pltpu.VMEM((B, D), jnp.float32),
            pltpu.VMEM((B, 1), jnp.float32),
            pltpu.VMEM((B, D), jnp.float32),
            pltpu.VMEM((B, 1), jnp.float32),
        ],
    )(degp5, u1, u2, s10, s11, s20, s21, bt1, bt2, w1t, b1, w2t, b2, fc1t,
      fc1b, fc2t, fc2b, f1, f2, f3, f4, fb, m1, m2)


def kernel(pro1_x, pro2_x, mas1_straight, mas1_flipped, mas2_straight,
           mas2_flipped, params, pro1_edge_index, pro1_batch,
           pro2_edge_index, pro2_batch):
    p = params
    ei1 = pro1_edge_index.astype(jnp.int32)
    ei2 = pro2_edge_index.astype(jnp.int32)
    src1 = ei1[0].reshape(E // CW, CW)
    dst1 = ei1[1].reshape(E // CW, CW)
    src2 = ei2[0].reshape(E // CW, CW)
    dst2 = ei2[1].reshape(E // CW, CW)
    bt1 = pro1_batch.astype(jnp.int32).reshape(NRB, 1, RB)
    bt2 = pro2_batch.astype(jnp.int32).reshape(NRB, 1, RB)
    ones_n = jnp.ones((N,), jnp.float32)

    degp = _deg_kernel()(dst1, dst2, ones_n)
    degp5 = degp.reshape(2, NC, NRB, 1, RB)
    u1, u2 = _scale_call(degp5, pro1_x, pro2_x)
    Sp = _agg_kernel()(u1, u2, src1, dst1, src2, dst2)

    ind = jnp.concatenate(
        [jnp.ones((B, L, 1), jnp.float32), jnp.zeros((B, L, 1), jnp.float32)],
        axis=1)
    x1aug = jnp.concatenate(
        [jnp.concatenate([mas1_straight, mas1_flipped], axis=1), ind],
        axis=-1)
    x2aug = jnp.concatenate(
        [jnp.concatenate([mas2_straight, mas2_flipped], axis=1), ind],
        axis=-1)
    Waug = jnp.zeros((DM, DESC + 1), jnp.float32)
    Waug = Waug.at[:TD, :DESC].set(p['red_W'])
    Waug = Waug.at[TD, DESC].set(1.0)
    baug = jnp.concatenate([p['red_b'], jnp.zeros((1,), jnp.float32)])
    stk = tuple(
        jnp.stack([p['t%d_%s' % (l, n)] for l in range(NL)])
        for n in ('Wqkv', 'bqkv', 'Wo', 'bo', 'ln1g', 'ln1b', 'W1', 'b1',
                  'W2', 'b2', 'ln2g', 'ln2b'))
    (WqkvS, bqkvS, WoS, boS, ln1gS, ln1bS, W1S, b1S, W2S, b2S, ln2gS,
     ln2bS) = stk
    WqkvT = WqkvS.transpose(0, 2, 1)
    m1, m2 = _tr_call(
        x1aug, x2aug, Waug.T, baug,
        (WqkvT[:, :, :DM], WqkvT[:, :, DM:2 * DM], WqkvT[:, :, 2 * DM:],
         bqkvS[:, :DM], bqkvS[:, DM:2 * DM], bqkvS[:, 2 * DM:],
         WoS.transpose(0, 2, 1), boS, ln1gS, ln1bS,
         W1S.transpose(0, 2, 1), b1S, W2S.transpose(0, 2, 1), b2S, ln2gS,
         ln2bS))

    fw = p['fin_W']
    out = _final_call(
        degp5, u1, u2, Sp[0, 0], Sp[0, 1], Sp[1, 0], Sp[1, 1], bt1, bt2,
        p['gcn1_W'].T, p['gcn1_b'], p['gcn2_W'].T, p['gcn2_b'],
        p['fc1_W'].T, p['fc1_b'], p['fc2_W'].T, p['fc2_b'],
        fw[:, :OUT].T, fw[:, OUT:2 * OUT].T,
        fw[:, 2 * OUT:2 * OUT + DM].T, fw[:, 2 * OUT + DM:].T,
        p['fin_b'], m1, m2)
    return out
